# Initial kernel scaffold; baseline (speedup 1.0000x reference)
import jax
import jax.numpy as jnp
from jax.experimental import pallas as pl


def kernel(y_pred, y_true):
    raise NotImplementedError("write your pallas kernel here")


# speedup vs baseline: 16.6756x; 16.6756x over previous
import math

import jax
import jax.numpy as jnp
from jax.experimental import pallas as pl

_EPS = 1e-6
_LOG_EPS = math.log(_EPS)
_LOG_1MEPS = math.log1p(-_EPS)
_K = 1000
_HARD_WEIGHT = 0.01
_INTERPRET = False

_ROWS_PER_BLOCK = 8


def _loss_body(x_ref, z_ref, t_ref, out_ref):
    x = x_ref[...]
    z = z_ref[...]
    t = t_ref[...][:, :1]
    lq = jax.nn.log_sigmoid(x)
    l1mq = jax.nn.log_sigmoid(-x)
    p = jnp.clip(jax.nn.sigmoid(z), _EPS, 1.0 - _EPS)
    ls = p * (jnp.log(p) - lq) + (1.0 - p) * (jnp.log1p(-p) - l1mq)
    col = jax.lax.broadcasted_iota(jnp.int32, x.shape, 1)
    kl0 = _EPS * (_LOG_EPS - lq) + (1.0 - _EPS) * (_LOG_1MEPS - l1mq)
    kl1 = (1.0 - _EPS) * (_LOG_1MEPS - lq) + _EPS * (_LOG_EPS - l1mq)
    lh = jnp.where(col == t, kl1, kl0)
    out_ref[...] = jnp.concatenate(
        [jnp.maximum(ls, 0.0), jnp.maximum(lh, 0.0)], axis=0)


def _topk_body(l_ref, out_ref):
    v = l_ref[...]
    bits = jax.lax.bitcast_convert_type(v, jnp.int32)
    r = v.shape[0]

    def step(_, carry):
        lo, hi = carry
        mid = lo + (hi - lo) // 2
        cnt = jnp.sum(jnp.where(bits > mid, 1, 0), axis=1, keepdims=True)
        take = cnt >= _K
        return (jnp.where(take, mid, lo), jnp.where(take, hi, mid))

    lo0 = jnp.full((r, 1), -1, jnp.int32)
    hi0 = jnp.full((r, 1), 0x7F800000, jnp.int32)
    lo, hi = jax.lax.fori_loop(0, 31, step, (lo0, hi0))
    thr = jax.lax.bitcast_convert_type(hi, jnp.float32)
    ge = bits >= hi
    s_ge = jnp.sum(jnp.where(ge, v, 0.0), axis=1, keepdims=True)
    c_ge = jnp.sum(jnp.where(ge, 1, 0), axis=1, keepdims=True)
    topk_sum = s_ge - (c_ge.astype(jnp.float32) - float(_K)) * thr
    out_ref[...] = jnp.broadcast_to(topk_sum, (r, 128))


def kernel(y_pred, y_true):
    y_pred = y_pred.astype(jnp.float32)
    half = y_pred.shape[0] // 2
    c = y_pred.shape[1]
    rb = _ROWS_PER_BLOCK
    nb = half // rb
    x = y_pred[:half]
    z = y_pred[half:]
    t = jnp.broadcast_to(
        y_true[half:].astype(jnp.int32)[:, None], (half, 128))

    losses = pl.pallas_call(
        _loss_body,
        grid=(nb,),
        in_specs=[
            pl.BlockSpec((rb, c), lambda i: (i, 0)),
            pl.BlockSpec((rb, c), lambda i: (i, 0)),
            pl.BlockSpec((rb, 128), lambda i: (i, 0)),
        ],
        out_specs=pl.BlockSpec((2 * rb, c), lambda i: (i, 0)),
        out_shape=jax.ShapeDtypeStruct((2 * half, c), jnp.float32),
        interpret=_INTERPRET,
    )(x, z, t)

    sums = pl.pallas_call(
        _topk_body,
        grid=(2 * nb,),
        in_specs=[pl.BlockSpec((rb, c), lambda i: (i, 0))],
        out_specs=pl.BlockSpec((rb, 128), lambda i: (i, 0)),
        out_shape=jax.ShapeDtypeStruct((2 * half, 128), jnp.float32),
        interpret=_INTERPRET,
    )(losses)

    per_row = sums[:, 0].reshape(nb, 2, rb)
    denom = float(half * _K)
    soft = jnp.sum(per_row[:, 0, :]) / denom
    hard = jnp.sum(per_row[:, 1, :]) / denom
    return soft + _HARD_WEIGHT * hard


# speedup vs baseline: 17.3986x; 1.0434x over previous
import math

import jax
import jax.numpy as jnp
from jax.experimental import pallas as pl

_EPS = 1e-6
_LOG_EPS = math.log(_EPS)
_LOG_1MEPS = math.log1p(-_EPS)
_K = 1000
_HARD_WEIGHT = 0.01
_INTERPRET = False

_ROWS_PER_BLOCK = 8


_C0 = _EPS * _LOG_EPS + (1.0 - _EPS) * _LOG_1MEPS


def _loss_body(x_ref, z_ref, t_ref, out_ref):
    x = x_ref[...]
    z = z_ref[...]
    t = t_ref[...][:, :1]
    ux = jnp.exp(-jnp.abs(x))
    sp_x = jnp.maximum(x, 0.0) + jnp.log1p(ux)
    uz = jnp.exp(-jnp.abs(z))
    sp_z = jnp.maximum(z, 0.0) + jnp.log1p(uz)
    vz = 1.0 / (1.0 + uz)
    s_z = jnp.where(z >= 0.0, vz, uz * vz)
    ls = sp_x - sp_z + s_z * (z - x)
    col = jax.lax.broadcasted_iota(jnp.int32, x.shape, 1)
    lh = (_C0 + sp_x - _EPS * x) - jnp.where(
        col == t, (1.0 - 2.0 * _EPS) * x, 0.0)
    out_ref[...] = jnp.concatenate(
        [jnp.maximum(ls, 0.0), jnp.maximum(lh, 0.0)], axis=0)


def _topk_body(l_ref, out_ref):
    v = l_ref[...]
    bits = jax.lax.bitcast_convert_type(v, jnp.int32)
    r = v.shape[0]

    def step(_, carry):
        lo, hi = carry
        mid = lo + (hi - lo) // 2
        cnt = jnp.sum(jnp.where(bits > mid, 1, 0), axis=1, keepdims=True)
        take = cnt >= _K
        return (jnp.where(take, mid, lo), jnp.where(take, hi, mid))

    lo0 = jnp.full((r, 1), -1, jnp.int32)
    hi0 = jnp.full((r, 1), 0x7F800000, jnp.int32)
    lo, hi = jax.lax.fori_loop(0, 31, step, (lo0, hi0))
    thr = jax.lax.bitcast_convert_type(hi, jnp.float32)
    ge = bits >= hi
    s_ge = jnp.sum(jnp.where(ge, v, 0.0), axis=1, keepdims=True)
    c_ge = jnp.sum(jnp.where(ge, 1, 0), axis=1, keepdims=True)
    topk_sum = s_ge - (c_ge.astype(jnp.float32) - float(_K)) * thr
    out_ref[...] = jnp.broadcast_to(topk_sum, (r, 128))


def kernel(y_pred, y_true):
    y_pred = y_pred.astype(jnp.float32)
    half = y_pred.shape[0] // 2
    c = y_pred.shape[1]
    rb = _ROWS_PER_BLOCK
    nb = half // rb
    x = y_pred[:half]
    z = y_pred[half:]
    t = jnp.broadcast_to(
        y_true[half:].astype(jnp.int32)[:, None], (half, 128))

    losses = pl.pallas_call(
        _loss_body,
        grid=(nb,),
        in_specs=[
            pl.BlockSpec((rb, c), lambda i: (i, 0)),
            pl.BlockSpec((rb, c), lambda i: (i, 0)),
            pl.BlockSpec((rb, 128), lambda i: (i, 0)),
        ],
        out_specs=pl.BlockSpec((2 * rb, c), lambda i: (i, 0)),
        out_shape=jax.ShapeDtypeStruct((2 * half, c), jnp.float32),
        interpret=_INTERPRET,
    )(x, z, t)

    sums = pl.pallas_call(
        _topk_body,
        grid=(2 * nb,),
        in_specs=[pl.BlockSpec((rb, c), lambda i: (i, 0))],
        out_specs=pl.BlockSpec((rb, 128), lambda i: (i, 0)),
        out_shape=jax.ShapeDtypeStruct((2 * half, 128), jnp.float32),
        interpret=_INTERPRET,
    )(losses)

    per_row = sums[:, 0].reshape(nb, 2, rb)
    denom = float(half * _K)
    soft = jnp.sum(per_row[:, 0, :]) / denom
    hard = jnp.sum(per_row[:, 1, :]) / denom
    return soft + _HARD_WEIGHT * hard
